# Initial kernel scaffold; baseline (speedup 1.0000x reference)
import jax
import jax.numpy as jnp
from jax.experimental import pallas as pl


def kernel(src_word, src_pos, emb_table):
    raise NotImplementedError("write your pallas kernel here")


# speedup vs baseline: 4.0604x; 4.0604x over previous
import functools

import jax
import jax.numpy as jnp
from jax import lax
from jax.experimental import pallas as pl
from jax.experimental.pallas import tpu as pltpu
from jax.experimental.pallas import tpu_sc as plsc

B, L, D = 4096, 200, 128
N = B * L
NC, NS, LANES = 2, 16, 16
NW = NC * NS
ROWS_PER_W = N // NW
CHUNK = 256
NCHUNK = ROWS_PER_W // CHUNK
IDX_W = 128
GPC = CHUNK // IDX_W
SCALE = float(D) ** 0.5


def _body(src_hbm, idx_hbm, tbl_hbm, out_hbm, idx_v, src_v, rows_v, sem):
    wid = lax.axis_index("s") * NC + lax.axis_index("c")
    base = wid * ROWS_PER_W
    pltpu.sync_copy(idx_hbm.at[pl.ds(wid * (ROWS_PER_W // IDX_W), ROWS_PER_W // IDX_W), :], idx_v)

    def step(c, carry):
        row0 = base + c * CHUNK
        in_cp = pltpu.async_copy(src_hbm.at[pl.ds(row0, CHUNK), :], src_v, sem)
        gathers = []
        for g in range(GPC):
            gathers.append(
                pltpu.async_copy(
                    tbl_hbm.at[idx_v.at[c * GPC + g]],
                    rows_v.at[pl.ds(g * IDX_W, IDX_W), :],
                    sem,
                )
            )
        in_cp.wait()
        for cp in gathers:
            cp.wait()

        def row_fma(i, c2):
            for j in range(D // LANES):
                sl = pl.ds(j * LANES, LANES)
                src_v[i, sl] = src_v[i, sl] * SCALE + rows_v[i, sl]
            return c2

        lax.fori_loop(0, CHUNK, row_fma, 0)

        pltpu.sync_copy(src_v, out_hbm.at[pl.ds(row0, CHUNK), :])
        return carry

    lax.fori_loop(0, NCHUNK, step, 0)


_sc_call = pl.kernel(
    _body,
    out_type=jax.ShapeDtypeStruct((N, D), jnp.float32),
    mesh=plsc.VectorSubcoreMesh(core_axis_name="c", subcore_axis_name="s"),
    scratch_types=[
        pltpu.VMEM((ROWS_PER_W // IDX_W, IDX_W), jnp.int32),
        pltpu.VMEM((CHUNK, D), jnp.float32),
        pltpu.VMEM((CHUNK, D), jnp.float32),
        pltpu.SemaphoreType.DMA,
    ],
)


@jax.jit
def kernel(src_word, src_pos, emb_table):
    src2 = src_word.reshape(N, D)
    idx = src_pos.astype(jnp.int32).reshape(N // IDX_W, IDX_W)
    out = _sc_call(src2, idx, emb_table)
    return out.reshape(B, L, D)


# speedup vs baseline: 5.6478x; 1.3910x over previous
import jax
import jax.numpy as jnp
from jax import lax
from jax.experimental import pallas as pl
from jax.experimental.pallas import tpu as pltpu
from jax.experimental.pallas import tpu_sc as plsc

B, L, D = 4096, 200, 128
N = B * L
NC, NS, LANES = 2, 16, 16
NW = NC * NS
ROWS_PER_W = N // NW
CHUNK = 256
NCHUNK = ROWS_PER_W // CHUNK
IDX_W = 128
GPC = CHUNK // IDX_W
SCALE = float(D) ** 0.5


def _body(src_hbm, idx_hbm, tbl_hbm, out_hbm,
          idx_v, buf0, buf1, sl0, sl1, sg0, sg1, ss0, ss1):
    wid = lax.axis_index("s") * NC + lax.axis_index("c")
    base = wid * ROWS_PER_W
    bufs = (buf0, buf1)
    sem_l = (sl0, sl1)
    sem_g = (sg0, sg1)
    sem_s = (ss0, ss1)

    pltpu.sync_copy(
        idx_hbm.at[pl.ds(wid * (ROWS_PER_W // IDX_W), ROWS_PER_W // IDX_W), :],
        idx_v)

    def src_slice(c):
        return src_hbm.at[pl.ds(base + c * CHUNK, CHUNK), :]

    def out_slice(c):
        return out_hbm.at[pl.ds(base + c * CHUNK, CHUNK), :]

    def scale_buf(buf):
        def row(i, acc):
            for j in range(D // LANES):
                sl = pl.ds(j * LANES, LANES)
                buf[i, sl] = buf[i, sl] * SCALE
            return acc
        lax.fori_loop(0, CHUNK, row, 0)

    def chunk_body(c, b, first):
        buf = bufs[b]
        pltpu.make_async_copy(src_slice(c), buf, sem_l[b]).wait()
        scale_buf(buf)
        gadds = [
            pltpu.async_copy(
                tbl_hbm.at[idx_v.at[c * GPC + g]],
                buf.at[pl.ds(g * IDX_W, IDX_W), :],
                sem_g[b],
                add=True,
            )
            for g in range(GPC)
        ]
        if not first:
            pltpu.make_async_copy(bufs[1 - b], out_slice(c - 1), sem_s[1 - b]).wait()

        def prefetch():
            pltpu.async_copy(src_slice(c + 1), bufs[1 - b], sem_l[1 - b])

        has_next = c + 1 < NCHUNK
        if isinstance(has_next, bool):
            if has_next:
                prefetch()
        else:
            pl.when(has_next)(prefetch)
        for cp in gadds:
            cp.wait()
        pltpu.async_copy(buf, out_slice(c), sem_s[b])

    pltpu.async_copy(src_slice(0), bufs[0], sem_l[0])
    chunk_body(0, 0, first=True)
    chunk_body(1, 1, first=False)

    def super_step(k, carry):
        chunk_body(2 * k, 0, first=False)
        chunk_body(2 * k + 1, 1, first=False)
        return carry

    lax.fori_loop(1, NCHUNK // 2, super_step, 0)
    pltpu.make_async_copy(bufs[1], out_slice(NCHUNK - 1), sem_s[1]).wait()


_sc_call = pl.kernel(
    _body,
    out_type=jax.ShapeDtypeStruct((N, D), jnp.float32),
    mesh=plsc.VectorSubcoreMesh(core_axis_name="c", subcore_axis_name="s"),
    scratch_types=[
        pltpu.VMEM((ROWS_PER_W // IDX_W, IDX_W), jnp.int32),
        pltpu.VMEM((CHUNK, D), jnp.float32),
        pltpu.VMEM((CHUNK, D), jnp.float32),
        pltpu.SemaphoreType.DMA,
        pltpu.SemaphoreType.DMA,
        pltpu.SemaphoreType.DMA,
        pltpu.SemaphoreType.DMA,
        pltpu.SemaphoreType.DMA,
        pltpu.SemaphoreType.DMA,
    ],
)


@jax.jit
def kernel(src_word, src_pos, emb_table):
    src2 = src_word.reshape(N, D)
    idx = src_pos.astype(jnp.int32).reshape(N // IDX_W, IDX_W)
    out = _sc_call(src2, idx, emb_table)
    return out.reshape(B, L, D)
